# Initial kernel scaffold; baseline (speedup 1.0000x reference)
import jax
import jax.numpy as jnp
from jax.experimental import pallas as pl


def kernel(x_t, dx_dt, edge_index, atomic_numbers, batch_ids, covalent_radii):
    raise NotImplementedError("write your pallas kernel here")


# speedup vs baseline: 109.6456x; 109.6456x over previous
import functools

import jax
import jax.numpy as jnp
from jax import lax
from jax.experimental import pallas as pl
from jax.experimental.pallas import tpu as pltpu
from jax.experimental.pallas import tpu_sc as plsc

N_NODES = 100000
N_EDGES = 3200000
ALPHA = 1.7
BETA = 0.01
DIST_CLAMP = 0.1

NC, NS, L = 2, 16, 16
NW = NC * NS
CHUNK = 128
CHUNKS_PER_TILE = 782
PER_TILE = CHUNK * CHUNKS_PER_TILE
E_PAD = PER_TILE * NW
COV_PAD = 128


def _rsqrt(d2):
    i = lax.bitcast_convert_type(d2, jnp.int32)
    i = jnp.int32(0x5F3759DF) - lax.shift_right_logical(i, 1)
    y = lax.bitcast_convert_type(i, jnp.float32)
    half = d2 * 0.5
    for _ in range(3):
        y = y * (1.5 - half * y * y)
    return y


def _sc_body(xdx_hbm, src_hbm, dst_hbm, an_hbm, cov_hbm, out_hbm,
             an_v, cov_v, src_v, dst_v, rows_s, rows_d, acc_v, sem):
    wid = lax.axis_index("s") * NC + lax.axis_index("c")
    tile_base = wid * PER_TILE

    pltpu.sync_copy(an_hbm, an_v)
    pltpu.sync_copy(cov_hbm, cov_v)
    acc_v[...] = jnp.zeros((L,), jnp.float32)

    iota = lax.iota(jnp.int32, L)

    @pl.loop(0, CHUNKS_PER_TILE)
    def _chunk(c):
        base = tile_base + c * CHUNK
        pltpu.sync_copy(src_hbm.at[pl.ds(base, CHUNK)], src_v)
        pltpu.sync_copy(dst_hbm.at[pl.ds(base, CHUNK)], dst_v)
        cp_s = pltpu.async_copy(xdx_hbm.at[src_v], rows_s, sem)
        cp_d = pltpu.async_copy(xdx_hbm.at[dst_v], rows_d, sem)
        cp_s.wait()
        cp_d.wait()

        for j in range(CHUNK // L):
            sl = pl.ds(j * L, L)
            s16 = src_v[sl]
            d16 = dst_v[sl]
            an_s = plsc.load_gather(an_v, [s16])
            an_d = plsc.load_gather(an_v, [d16])
            r = (plsc.load_gather(cov_v, [an_s])
                 + plsc.load_gather(cov_v, [an_d]))
            row = iota + (j * L)

            def col(rows, c_):
                return plsc.load_gather(rows, [row, jnp.full((L,), c_, jnp.int32)])

            dx0 = col(rows_s, 0) - col(rows_d, 0)
            dx1 = col(rows_s, 1) - col(rows_d, 1)
            dx2 = col(rows_s, 2) - col(rows_d, 2)
            dv0 = col(rows_s, 3) - col(rows_d, 3)
            dv1 = col(rows_s, 4) - col(rows_d, 4)
            dv2 = col(rows_s, 5) - col(rows_d, 5)

            d2 = dx0 * dx0 + dx1 * dx1 + dx2 * dx2
            dot = dx0 * dv0 + dx1 * dv1 + dx2 * dv2
            y = _rsqrt(d2)
            dist = d2 * y
            rinv = 1.0 / r
            eterm = jnp.exp(ALPHA - (ALPHA * dist) * rinv)
            g = -ALPHA * rinv * eterm - jnp.where(
                dist > DIST_CLAMP, (BETA * r) * (y * y), 0.0)
            jvp = g * dot * y
            valid = (base + (j * L) + iota) < N_EDGES
            jvp = jnp.where(valid, jvp, 0.0)
            acc_v[...] = acc_v[...] + jvp * jvp

    pltpu.sync_copy(acc_v, out_hbm.at[wid])


@jax.jit
def _sc_partials(xdx, src_p, dst_p, an, cov_p):
    mesh = plsc.VectorSubcoreMesh(core_axis_name="c", subcore_axis_name="s")
    fn = pl.kernel(
        _sc_body,
        out_type=jax.ShapeDtypeStruct((NW, L), jnp.float32),
        mesh=mesh,
        compiler_params=pltpu.CompilerParams(
            needs_layout_passes=False, use_tc_tiling_on_sc=False),
        scratch_types=[
            pltpu.VMEM((N_NODES,), jnp.int32),
            pltpu.VMEM((COV_PAD,), jnp.float32),
            pltpu.VMEM((CHUNK,), jnp.int32),
            pltpu.VMEM((CHUNK,), jnp.int32),
            pltpu.VMEM((CHUNK, 8), jnp.float32),
            pltpu.VMEM((CHUNK, 8), jnp.float32),
            pltpu.VMEM((L,), jnp.float32),
            pltpu.SemaphoreType.DMA,
        ],
    )
    return fn(xdx, src_p, dst_p, an, cov_p)


def kernel(x_t, dx_dt, edge_index, atomic_numbers, batch_ids, covalent_radii):
    edge_index = edge_index.astype(jnp.int32)
    src = edge_index[0]
    dst = edge_index[1]
    pad = E_PAD - N_EDGES
    src_p = jnp.concatenate([src, jnp.zeros((pad,), jnp.int32)])
    dst_p = jnp.concatenate([dst, jnp.ones((pad,), jnp.int32)])
    xdx = jnp.concatenate(
        [x_t.astype(jnp.float32), dx_dt.astype(jnp.float32),
         jnp.zeros((N_NODES, 2), jnp.float32)], axis=1)
    cov_p = jnp.concatenate(
        [covalent_radii.astype(jnp.float32),
         jnp.zeros((COV_PAD - covalent_radii.shape[0],), jnp.float32)])
    partials = _sc_partials(xdx, src_p, dst_p,
                            atomic_numbers.astype(jnp.int32), cov_p)
    num_segments = batch_ids[-1].astype(jnp.int32) + 1
    return jnp.sum(partials) / num_segments


# speedup vs baseline: 338.4756x; 3.0870x over previous
import jax
import jax.numpy as jnp
from jax import lax
from jax.experimental import pallas as pl
from jax.experimental.pallas import tpu as pltpu
from jax.experimental.pallas import tpu_sc as plsc

N_NODES = 100000
N_EDGES = 3200000
ALPHA = 1.7
BETA = 0.01
DIST_CLAMP = 0.1

NC, NS, L = 2, 16, 16
NW = NC * NS
CHUNK = 128
CPT = 784
PER_TILE = CHUNK * CPT
E_PAD = PER_TILE * NW
TOTAL_CHUNKS = CPT * NW
COV_PAD = 128
ID = 8
RD = 4

PAD_SRC = N_NODES
PAD_DST = N_NODES + 1
N_ROWS = N_NODES + 2
NR_PAD = N_NODES + 16
PAD_BIG = 1.0e3
AN_BLK = 4000


def _rsqrt(d2):
    i = lax.bitcast_convert_type(d2, jnp.int32)
    i = jnp.int32(0x5F3759DF) - lax.shift_right_logical(i, 1)
    y = lax.bitcast_convert_type(i, jnp.float32)
    half = d2 * 0.5
    for _ in range(2):
        y = y * (1.5 - half * y * y)
    return y


def _sc_body(xdx_hbm, src_hbm, dst_hbm, an_hbm, cov_hbm, out_hbm,
             r_v, cov_v, anb_v, srcb, dstb, rows_s, rows_d, acc_v, si, sr):
    wid = lax.axis_index("s") * NC + lax.axis_index("c")
    chunk0 = wid * CPT

    pltpu.sync_copy(cov_hbm, cov_v)
    iota = lax.iota(jnp.int32, L)

    @pl.loop(0, N_NODES // AN_BLK)
    def _rblk(blk):
        pltpu.sync_copy(an_hbm.at[pl.ds(blk * AN_BLK, AN_BLK)], anb_v)

        @pl.loop(0, AN_BLK // L)
        def _rgrp(j):
            idx = anb_v[pl.ds(j * L, L)]
            r_v[pl.ds(blk * AN_BLK + j * L, L)] = plsc.load_gather(cov_v, [idx])

    r_v[pl.ds(N_NODES, L)] = jnp.full((L,), 1.0, jnp.float32)

    def issue_idx(t, b):
        pltpu.async_copy(src_hbm.at[chunk0 + t], srcb.at[b], si[b])
        pltpu.async_copy(dst_hbm.at[chunk0 + t], dstb.at[b], si[b])

    def wait_idx(t, b):
        pltpu.make_async_copy(src_hbm.at[chunk0 + t], srcb.at[b], si[b]).wait()
        pltpu.make_async_copy(dst_hbm.at[chunk0 + t], dstb.at[b], si[b]).wait()

    def issue_rows(b, rb):
        pltpu.async_copy(xdx_hbm.at[srcb.at[b]], rows_s.at[rb], sr[rb])
        pltpu.async_copy(xdx_hbm.at[dstb.at[b]], rows_d.at[rb], sr[rb])

    def wait_rows(b, rb):
        pltpu.make_async_copy(xdx_hbm.at[srcb.at[b]], rows_s.at[rb], sr[rb]).wait()
        pltpu.make_async_copy(xdx_hbm.at[dstb.at[b]], rows_d.at[rb], sr[rb]).wait()

    for t in range(ID):
        issue_idx(t, t)
    for t in range(RD):
        wait_idx(t, t)
        issue_rows(t, t)

    def _blk(i, acc):
        c = i * ID
        for b in range(ID):
            cur = c + b
            rb = b % RD
            wait_rows(b, rb)

            for j in range(CHUNK // L):
                sl = pl.ds(j * L, L)
                s16 = srcb[b, sl]
                d16 = dstb[b, sl]
                r = (plsc.load_gather(r_v, [s16])
                     + plsc.load_gather(r_v, [d16]))
                row = iota + (j * L)

                def col(rows, c_):
                    return plsc.load_gather(
                        rows.at[rb], [row, jnp.full((L,), c_, jnp.int32)])

                dx0 = col(rows_s, 0) - col(rows_d, 0)
                dx1 = col(rows_s, 1) - col(rows_d, 1)
                dx2 = col(rows_s, 2) - col(rows_d, 2)
                dv0 = col(rows_s, 3) - col(rows_d, 3)
                dv1 = col(rows_s, 4) - col(rows_d, 4)
                dv2 = col(rows_s, 5) - col(rows_d, 5)

                d2 = dx0 * dx0 + dx1 * dx1 + dx2 * dx2
                dot = dx0 * dv0 + dx1 * dv1 + dx2 * dv2
                y = _rsqrt(d2)
                dist = d2 * y
                rinv = 1.0 / r
                eterm = jnp.exp(ALPHA - (ALPHA * dist) * rinv)
                g = -ALPHA * rinv * eterm - jnp.where(
                    dist > DIST_CLAMP, (BETA * r) * (y * y), 0.0)
                jvp = g * dot * y
                acc = acc + jvp * jvp

            @pl.when(cur + ID < CPT)
            def _():
                issue_idx(cur + ID, b)

            @pl.when(cur + RD < CPT)
            def _():
                b4 = (b + RD) % ID
                wait_idx(cur + RD, b4)
                issue_rows(b4, rb)
        return acc

    acc = lax.fori_loop(0, CPT // ID, _blk, jnp.zeros((L,), jnp.float32))
    acc_v[...] = acc
    pltpu.sync_copy(acc_v, out_hbm.at[wid])


@jax.jit
def _sc_partials(xdx, src_p, dst_p, an, cov_p):
    mesh = plsc.VectorSubcoreMesh(core_axis_name="c", subcore_axis_name="s")
    fn = pl.kernel(
        _sc_body,
        out_type=jax.ShapeDtypeStruct((NW, L), jnp.float32),
        mesh=mesh,
        compiler_params=pltpu.CompilerParams(
            needs_layout_passes=False, use_tc_tiling_on_sc=False),
        scratch_types=[
            pltpu.VMEM((NR_PAD,), jnp.float32),
            pltpu.VMEM((COV_PAD,), jnp.float32),
            pltpu.VMEM((AN_BLK,), jnp.int32),
            pltpu.VMEM((ID, CHUNK), jnp.int32),
            pltpu.VMEM((ID, CHUNK), jnp.int32),
            pltpu.VMEM((RD, CHUNK, 8), jnp.float32),
            pltpu.VMEM((RD, CHUNK, 8), jnp.float32),
            pltpu.VMEM((L,), jnp.float32),
            [pltpu.SemaphoreType.DMA] * ID,
            [pltpu.SemaphoreType.DMA] * RD,
        ],
    )
    return fn(xdx, src_p, dst_p, an, cov_p)


def kernel(x_t, dx_dt, edge_index, atomic_numbers, batch_ids, covalent_radii):
    edge_index = edge_index.astype(jnp.int32)
    src = edge_index[0]
    dst = edge_index[1]
    pad = E_PAD - N_EDGES
    src_p = jnp.concatenate(
        [src, jnp.full((pad,), PAD_SRC, jnp.int32)]).reshape(TOTAL_CHUNKS, CHUNK)
    dst_p = jnp.concatenate(
        [dst, jnp.full((pad,), PAD_DST, jnp.int32)]).reshape(TOTAL_CHUNKS, CHUNK)
    sentinel = jnp.zeros((2, 8), jnp.float32).at[1, 0].set(PAD_BIG)
    xdx = jnp.concatenate(
        [jnp.concatenate(
            [x_t.astype(jnp.float32), dx_dt.astype(jnp.float32),
             jnp.zeros((N_NODES, 2), jnp.float32)], axis=1),
         sentinel], axis=0)
    cov_p = jnp.concatenate(
        [covalent_radii.astype(jnp.float32),
         jnp.zeros((COV_PAD - covalent_radii.shape[0],), jnp.float32)])
    partials = _sc_partials(xdx, src_p, dst_p,
                            atomic_numbers.astype(jnp.int32), cov_p)
    num_segments = batch_ids[-1].astype(jnp.int32) + 1
    return jnp.sum(partials) / num_segments
